# baseline (speedup 1.0000x reference)
import functools

import jax
import jax.numpy as jnp
from jax import lax
from jax.experimental import pallas as pl
from jax.experimental.pallas import tpu as pltpu
from jax.experimental.pallas import tpu_sc as plsc

D = 1024
F = 2816
H = 16
DH = 64
EPS = 1e-6
CAP = 0.5
TB = 512
FB = 1408


def _qkv_kern(x_ref, s_ref, wq_ref, wk_ref, wv_ref, q_ref, k_ref, v_ref):
    x = x_ref[:]
    var = jnp.mean(jnp.square(x), axis=-1, keepdims=True)
    xn = (x * lax.rsqrt(var + EPS)) * s_ref[:]
    q_ref[:] = jnp.dot(xn, wq_ref[:], preferred_element_type=jnp.float32)
    k_ref[:] = jnp.dot(xn, wk_ref[:], preferred_element_type=jnp.float32)
    v_ref[:] = jnp.dot(xn, wv_ref[:], preferred_element_type=jnp.float32)


def _qkv(x, s, wq, wk, wv):
    T = x.shape[0]
    out = jax.ShapeDtypeStruct((T, D), jnp.float32)
    return pl.pallas_call(
        _qkv_kern,
        grid=(T // TB,),
        in_specs=[
            pl.BlockSpec((TB, D), lambda i: (i, 0)),
            pl.BlockSpec((1, D), lambda i: (0, 0)),
            pl.BlockSpec((D, D), lambda i: (0, 0)),
            pl.BlockSpec((D, D), lambda i: (0, 0)),
            pl.BlockSpec((D, D), lambda i: (0, 0)),
        ],
        out_specs=[pl.BlockSpec((TB, D), lambda i: (i, 0))] * 3,
        out_shape=[out, out, out],
    )(x, s, wq, wk, wv)


def _attn_kern(q_ref, k_ref, v_ref, o_ref):
    qb = pl.program_id(1)
    lane = lax.broadcasted_iota(jnp.int32, (TB, 2 * DH), 1)
    q2 = q_ref[:]

    def body(kb, carry):
        m0, m1, l0, l1, acc = carry
        start = pl.multiple_of(kb * TB, TB)
        k2 = k_ref[pl.ds(start, TB), :]
        v2 = v_ref[pl.ds(start, TB), :]
        k_lo = jnp.where(lane < DH, k2, 0.0)
        k_hi = jnp.where(lane >= DH, k2, 0.0)
        k_bd = jnp.concatenate([k_lo, k_hi], axis=0)
        v_lo = jnp.where(lane < DH, v2, 0.0)
        v_hi = jnp.where(lane >= DH, v2, 0.0)
        v_bd = jnp.concatenate([v_lo, v_hi], axis=0)
        s = lax.dot_general(q2, k_bd, (((1,), (1,)), ((), ())),
                            preferred_element_type=jnp.float32) * 0.125
        row = qb * TB + lax.broadcasted_iota(jnp.int32, (TB, TB), 0)
        col = start + lax.broadcasted_iota(jnp.int32, (TB, TB), 1)
        causal = row >= col
        s0 = jnp.where(causal, s[:, :TB], -1e9)
        s1 = jnp.where(causal, s[:, TB:], -1e9)
        m0n = jnp.maximum(m0, jnp.max(s0, axis=1, keepdims=True))
        m1n = jnp.maximum(m1, jnp.max(s1, axis=1, keepdims=True))
        p = jnp.concatenate([jnp.exp(s0 - m0n), jnp.exp(s1 - m1n)], axis=1)
        a0 = jnp.exp(m0 - m0n)
        a1 = jnp.exp(m1 - m1n)
        l0n = l0 * a0 + jnp.sum(p[:, :TB], axis=1, keepdims=True)
        l1n = l1 * a1 + jnp.sum(p[:, TB:], axis=1, keepdims=True)
        alpha = jnp.where(lane < DH, jnp.broadcast_to(a0, (TB, 2 * DH)),
                          jnp.broadcast_to(a1, (TB, 2 * DH)))
        acc_n = acc * alpha + jnp.dot(p, v_bd,
                                      preferred_element_type=jnp.float32)
        return m0n, m1n, l0n, l1n, acc_n

    mi = jnp.full((TB, 1), -1e30, jnp.float32)
    li = jnp.zeros((TB, 1), jnp.float32)
    acc0 = jnp.zeros((TB, 2 * DH), jnp.float32)
    m0, m1, l0, l1, acc = lax.fori_loop(0, qb + 1, body,
                                        (mi, mi, li, li, acc0))
    linv = jnp.where(lane < DH, jnp.broadcast_to(l0, (TB, 2 * DH)),
                     jnp.broadcast_to(l1, (TB, 2 * DH)))
    o_ref[:] = acc / linv


def _attention(q, k, v):
    T = q.shape[0]
    return pl.pallas_call(
        _attn_kern,
        grid=(H // 2, T // TB),
        in_specs=[
            pl.BlockSpec((TB, 2 * DH), lambda h, i: (i, h)),
            pl.BlockSpec((T, 2 * DH), lambda h, i: (0, h)),
            pl.BlockSpec((T, 2 * DH), lambda h, i: (0, h)),
        ],
        out_specs=pl.BlockSpec((TB, 2 * DH), lambda h, i: (i, h)),
        out_shape=jax.ShapeDtypeStruct((T, D), jnp.float32),
    )(q, k, v)


def _matmul_add_kern(a_ref, w_ref, r_ref, o_ref):
    o_ref[:] = r_ref[:] + jnp.dot(a_ref[:], w_ref[:],
                                  preferred_element_type=jnp.float32)


def _matmul_add(a, w, r):
    T = a.shape[0]
    N = w.shape[1]
    return pl.pallas_call(
        _matmul_add_kern,
        grid=(T // TB,),
        in_specs=[
            pl.BlockSpec((TB, D), lambda i: (i, 0)),
            pl.BlockSpec((D, N), lambda i: (0, 0)),
            pl.BlockSpec((TB, N), lambda i: (i, 0)),
        ],
        out_specs=pl.BlockSpec((TB, N), lambda i: (i, 0)),
        out_shape=jax.ShapeDtypeStruct((T, N), jnp.float32),
    )(a, w, r)


def _mlp_kern(x_ref, s_ref, wg_ref, wu_ref, wd_ref, o_ref):
    fb = pl.program_id(1)
    x = x_ref[:]
    var = jnp.mean(jnp.square(x), axis=-1, keepdims=True)
    xn = (x * lax.rsqrt(var + EPS)) * s_ref[:]
    g = jnp.dot(xn, wg_ref[:], preferred_element_type=jnp.float32)
    u = jnp.dot(xn, wu_ref[:], preferred_element_type=jnp.float32)
    t = (g * jax.nn.sigmoid(g)) * u
    part = jnp.dot(t, wd_ref[:], preferred_element_type=jnp.float32)

    @pl.when(fb == 0)
    def _():
        o_ref[:] = x + part

    @pl.when(fb != 0)
    def _():
        o_ref[:] = o_ref[:] + part


def _mlp_residual(x, s, wg, wu, wd):
    T = x.shape[0]
    return pl.pallas_call(
        _mlp_kern,
        grid=(T // TB, F // FB),
        in_specs=[
            pl.BlockSpec((TB, D), lambda i, j: (i, 0)),
            pl.BlockSpec((1, D), lambda i, j: (0, 0)),
            pl.BlockSpec((D, FB), lambda i, j: (0, j)),
            pl.BlockSpec((D, FB), lambda i, j: (0, j)),
            pl.BlockSpec((FB, D), lambda i, j: (j, 0)),
        ],
        out_specs=pl.BlockSpec((TB, D), lambda i, j: (i, 0)),
        out_shape=jax.ShapeDtypeStruct((T, D), jnp.float32),
    )(x, s, wg, wu, wd)


def _score_kern(proc_ref, s_ref, wg_ref, wu_ref, wd_ref, orig_ref,
                bce_ref, bcu_ref, g_ref, last_row, prev_s, pred_acc):
    i = pl.program_id(0)
    fb = pl.program_id(1)
    nfb = pl.num_programs(1)

    @pl.when((i == 0) & (fb == 0))
    def _():
        last_row[:] = jnp.zeros((1, D), jnp.float32)

    @pl.when(fb == 0)
    def _():
        proc = proc_ref[:]
        prev_s[:] = jnp.concatenate([last_row[:], proc[:TB - 1]], axis=0)
        last_row[:] = proc[TB - 1:TB]

    x = prev_s[:]
    var = jnp.mean(jnp.square(x), axis=-1, keepdims=True)
    xn = (x * lax.rsqrt(var + EPS)) * s_ref[:]
    g = jnp.dot(xn, wg_ref[:], preferred_element_type=jnp.float32)
    u = jnp.dot(xn, wu_ref[:], preferred_element_type=jnp.float32)
    t = (g * jax.nn.sigmoid(g)) * u
    part = jnp.dot(t, wd_ref[:], preferred_element_type=jnp.float32)

    @pl.when(fb == 0)
    def _():
        pred_acc[:] = part

    @pl.when(fb != 0)
    def _():
        pred_acc[:] = pred_acc[:] + part

    @pl.when(fb == nfb - 1)
    def _():
        pred = pred_acc[:]
        res = proc_ref[:] - orig_ref[:]
        d_st = jnp.sum(res * res, axis=-1, keepdims=True) * (1.0 / D)
        e = res - pred
        d_ch = jnp.sum(e * e, axis=-1, keepdims=True) * (1.0 / D)
        logit = bce_ref[0, 0] * d_st - bcu_ref[0, 0] * d_ch
        g_ref[:] = jax.nn.sigmoid(logit)


def _tn_scores(proc, s, wg, wu, wd, orig, bce, bcu):
    T = proc.shape[0]
    return pl.pallas_call(
        _score_kern,
        grid=(T // TB, F // FB),
        in_specs=[
            pl.BlockSpec((TB, D), lambda i, j: (i, 0)),
            pl.BlockSpec((1, D), lambda i, j: (0, 0)),
            pl.BlockSpec((D, FB), lambda i, j: (0, j)),
            pl.BlockSpec((D, FB), lambda i, j: (0, j)),
            pl.BlockSpec((FB, D), lambda i, j: (j, 0)),
            pl.BlockSpec((TB, D), lambda i, j: (i, 0)),
            pl.BlockSpec((1, 1), lambda i, j: (0, 0)),
            pl.BlockSpec((1, 1), lambda i, j: (0, 0)),
        ],
        out_specs=pl.BlockSpec((TB, 1), lambda i, j: (i, 0)),
        out_shape=jax.ShapeDtypeStruct((T, 1), jnp.float32),
        scratch_shapes=[
            pltpu.VMEM((1, D), jnp.float32),
            pltpu.VMEM((TB, D), jnp.float32),
            pltpu.VMEM((TB, D), jnp.float32),
        ],
    )(proc, s, wg, wu, wd, orig, bce, bcu)


_SC_LANES = 16


def _sc_gate_body(g_hbm, out_hbm, g_v, out_v, acc_ref, idx_ref, sem, *, T, k):
    L = _SC_LANES
    nslice = T // L
    info = plsc.get_sparse_core_info()
    nc = info.num_cores
    wid = lax.axis_index("s") * nc + lax.axis_index("c")
    nw = nc * info.num_subcores
    per_w = T // nw
    slices_per_w = per_w // L

    zero_v = jnp.zeros((L,), jnp.int32)
    one_v = jnp.ones((L,), jnp.int32)
    k_v = jnp.full((L,), k, jnp.int32)
    iota_v = jnp.arange(L, dtype=jnp.int32)
    lstep_v = jnp.full((L,), L, jnp.int32)

    pltpu.sync_copy(g_hbm, g_v)

    def lane_total_v(vec):
        tot = vec[0]
        for i in range(1, L):
            tot = tot + vec[i]
        return jnp.full((L,), tot, jnp.int32)

    def count_ge(mid_v):
        acc_ref[...] = zero_v

        def body(j, c):
            b = lax.bitcast_convert_type(g_v[pl.ds(j * L, L)], jnp.int32)
            acc_ref[...] = acc_ref[...] + jnp.where(b >= mid_v, one_v, zero_v)
            return c

        lax.fori_loop(0, nslice, body, jnp.int32(0))
        return lane_total_v(acc_ref[...])

    lo_v = zero_v
    hi_v = jnp.full((L,), 0x3F800000, jnp.int32)
    for _ in range(31):
        mid_v = lo_v + ((hi_v - lo_v + one_v) >> one_v)
        ge_k = count_ge(mid_v) >= k_v
        lo_v = jnp.where(ge_k, mid_v, lo_v)
        hi_v = jnp.where(ge_k, hi_v, mid_v - one_v)
    thr_v = lo_v
    n_tie_sel_v = k_v - count_ge(thr_v + one_v)

    def count_tie_lt(m_v):
        acc_ref[...] = zero_v
        idx_ref[...] = iota_v

        def body(j, c):
            b = lax.bitcast_convert_type(g_v[pl.ds(j * L, L)], jnp.int32)
            m = (b == thr_v) & (idx_ref[...] < m_v)
            acc_ref[...] = acc_ref[...] + jnp.where(m, one_v, zero_v)
            idx_ref[...] = idx_ref[...] + lstep_v
            return c

        lax.fori_loop(0, nslice, body, jnp.int32(0))
        return lane_total_v(acc_ref[...])

    tlo_v = zero_v
    thi_v = jnp.full((L,), T, jnp.int32)
    for _ in range(12):
        tmid_v = tlo_v + ((thi_v - tlo_v + one_v) >> one_v)
        ok = count_tie_lt(tmid_v) <= n_tie_sel_v
        tlo_v = jnp.where(ok, tmid_v, tlo_v)
        thi_v = jnp.where(ok, thi_v, tmid_v - one_v)
    m_v = tlo_v

    s0 = wid * slices_per_w
    for t in range(slices_per_w):
        vals = g_v[pl.ds((s0 + t) * L, L)]
        bits = lax.bitcast_convert_type(vals, jnp.int32)
        own_idx = jnp.full((L,), (s0 + t) * L, jnp.int32) + iota_v
        sel = (bits > thr_v) | ((bits == thr_v) & (own_idx < m_v))
        out_v[pl.ds(t * L, L)] = jnp.where(sel, vals,
                                           jnp.zeros((L,), jnp.float32))

    pltpu.sync_copy(out_v, out_hbm.at[pl.ds(wid * per_w, per_w)])


def _sc_gate(g, k):
    T = g.shape[0]
    mesh = plsc.VectorSubcoreMesh(core_axis_name="c", subcore_axis_name="s")
    fn = functools.partial(
        pl.kernel,
        mesh=mesh,
        out_type=jax.ShapeDtypeStruct((T,), jnp.float32),
        scratch_types=[
            pltpu.VMEM((T,), jnp.float32),
            pltpu.VMEM((T // 32,), jnp.float32),
            pltpu.VMEM((_SC_LANES,), jnp.int32),
            pltpu.VMEM((_SC_LANES,), jnp.int32),
            pltpu.SemaphoreType.DMA,
        ],
    )(functools.partial(_sc_gate_body, T=T, k=k))
    return fn(g)


def _combine_kern(gate_ref, orig_ref, proc_ref, o_ref):
    orig = orig_ref[:]
    o_ref[:] = orig + gate_ref[:] * (proc_ref[:] - orig)


def _combine(gate, orig, proc):
    T = orig.shape[0]
    return pl.pallas_call(
        _combine_kern,
        grid=(T // TB,),
        in_specs=[
            pl.BlockSpec((TB, 1), lambda i: (i, 0)),
            pl.BlockSpec((TB, D), lambda i: (i, 0)),
            pl.BlockSpec((TB, D), lambda i: (i, 0)),
        ],
        out_specs=pl.BlockSpec((TB, D), lambda i: (i, 0)),
        out_shape=jax.ShapeDtypeStruct((T, D), jnp.float32),
    )(gate, orig, proc)


def kernel(hidden_states, beta_ce, beta_cu, ln1, wq, wk, wv, wo, ln2,
           wg, wu, wd, tn_norm, tn_g, tn_u, tn_d):
    B, T, _ = hidden_states.shape
    x = hidden_states.reshape(T, D)

    q, k, v = _qkv(x, ln1.reshape(1, D), wq, wk, wv)
    o = _attention(q, k, v)
    h1 = _matmul_add(o, wo, x)

    proc = _mlp_residual(h1, ln2.reshape(1, D), wg, wu, wd)

    g_cont = _tn_scores(proc, tn_norm.reshape(1, D), tn_g, tn_u, tn_d,
                        x, beta_ce.reshape(1, 1), beta_cu.reshape(1, 1))

    kk = max(1, int(T * CAP))
    gate = _sc_gate(g_cont.reshape(T), kk)
    final = _combine(gate.reshape(T, 1), x, proc)
    return final.reshape(B, T, D)


# speedup vs baseline: 1.0510x; 1.0510x over previous
import functools

import jax
import jax.numpy as jnp
from jax import lax
from jax.experimental import pallas as pl
from jax.experimental.pallas import tpu as pltpu
from jax.experimental.pallas import tpu_sc as plsc

D = 1024
F = 2816
H = 16
DH = 64
EPS = 1e-6
CAP = 0.5
TB = 512
FB = 1408


def _qkv_kern(x_ref, s_ref, wq_ref, wk_ref, wv_ref, q_ref, k_ref, v_ref):
    s = s_ref[:]
    half = TB // 2
    for h in range(2):
        rows = pl.ds(h * half, half)
        x = x_ref[rows, :]
        var = jnp.mean(jnp.square(x), axis=-1, keepdims=True)
        xn = (x * lax.rsqrt(var + EPS)) * s
        q_ref[rows, :] = jnp.dot(xn, wq_ref[:],
                                 preferred_element_type=jnp.float32)
        k_ref[rows, :] = jnp.dot(xn, wk_ref[:],
                                 preferred_element_type=jnp.float32)
        v_ref[rows, :] = jnp.dot(xn, wv_ref[:],
                                 preferred_element_type=jnp.float32)


def _qkv(x, s, wq, wk, wv):
    T = x.shape[0]
    out = jax.ShapeDtypeStruct((T, D), jnp.float32)
    return pl.pallas_call(
        _qkv_kern,
        grid=(T // TB,),
        in_specs=[
            pl.BlockSpec((TB, D), lambda i: (i, 0)),
            pl.BlockSpec((1, D), lambda i: (0, 0)),
            pl.BlockSpec((D, D), lambda i: (0, 0)),
            pl.BlockSpec((D, D), lambda i: (0, 0)),
            pl.BlockSpec((D, D), lambda i: (0, 0)),
        ],
        out_specs=[pl.BlockSpec((TB, D), lambda i: (i, 0))] * 3,
        out_shape=[out, out, out],
    )(x, s, wq, wk, wv)


def _attn_kern(q_ref, k_ref, v_ref, o_ref):
    qb = pl.program_id(1)
    lane = lax.broadcasted_iota(jnp.int32, (TB, 2 * DH), 1)
    q2 = q_ref[:]

    def body(kb, carry):
        m0, m1, l0, l1, acc = carry
        start = pl.multiple_of(kb * TB, TB)
        k2 = k_ref[pl.ds(start, TB), :]
        v2 = v_ref[pl.ds(start, TB), :]
        k_lo = jnp.where(lane < DH, k2, 0.0)
        k_hi = jnp.where(lane >= DH, k2, 0.0)
        k_bd = jnp.concatenate([k_lo, k_hi], axis=0)
        v_lo = jnp.where(lane < DH, v2, 0.0)
        v_hi = jnp.where(lane >= DH, v2, 0.0)
        v_bd = jnp.concatenate([v_lo, v_hi], axis=0)
        s = lax.dot_general(q2, k_bd, (((1,), (1,)), ((), ())),
                            preferred_element_type=jnp.float32) * 0.125
        row = qb * TB + lax.broadcasted_iota(jnp.int32, (TB, TB), 0)
        col = start + lax.broadcasted_iota(jnp.int32, (TB, TB), 1)
        causal = row >= col
        s0 = jnp.where(causal, s[:, :TB], -1e9)
        s1 = jnp.where(causal, s[:, TB:], -1e9)
        m0n = jnp.maximum(m0, jnp.max(s0, axis=1, keepdims=True))
        m1n = jnp.maximum(m1, jnp.max(s1, axis=1, keepdims=True))
        p = jnp.concatenate([jnp.exp(s0 - m0n), jnp.exp(s1 - m1n)], axis=1)
        a0 = jnp.exp(m0 - m0n)
        a1 = jnp.exp(m1 - m1n)
        l0n = l0 * a0 + jnp.sum(p[:, :TB], axis=1, keepdims=True)
        l1n = l1 * a1 + jnp.sum(p[:, TB:], axis=1, keepdims=True)
        alpha = jnp.where(lane < DH, jnp.broadcast_to(a0, (TB, 2 * DH)),
                          jnp.broadcast_to(a1, (TB, 2 * DH)))
        acc_n = acc * alpha + jnp.dot(p, v_bd,
                                      preferred_element_type=jnp.float32)
        return m0n, m1n, l0n, l1n, acc_n

    mi = jnp.full((TB, 1), -1e30, jnp.float32)
    li = jnp.zeros((TB, 1), jnp.float32)
    acc0 = jnp.zeros((TB, 2 * DH), jnp.float32)
    m0, m1, l0, l1, acc = lax.fori_loop(0, qb + 1, body,
                                        (mi, mi, li, li, acc0))
    linv = jnp.where(lane < DH, jnp.broadcast_to(l0, (TB, 2 * DH)),
                     jnp.broadcast_to(l1, (TB, 2 * DH)))
    o_ref[:] = acc / linv


def _attention(q, k, v):
    T = q.shape[0]
    return pl.pallas_call(
        _attn_kern,
        grid=(H // 2, T // TB),
        in_specs=[
            pl.BlockSpec((TB, 2 * DH), lambda h, i: (i, h)),
            pl.BlockSpec((T, 2 * DH), lambda h, i: (0, h)),
            pl.BlockSpec((T, 2 * DH), lambda h, i: (0, h)),
        ],
        out_specs=pl.BlockSpec((TB, 2 * DH), lambda h, i: (i, h)),
        out_shape=jax.ShapeDtypeStruct((T, D), jnp.float32),
    )(q, k, v)


def _matmul_add_kern(a_ref, w_ref, r_ref, o_ref):
    o_ref[:] = r_ref[:] + jnp.dot(a_ref[:], w_ref[:],
                                  preferred_element_type=jnp.float32)


def _matmul_add(a, w, r):
    T = a.shape[0]
    N = w.shape[1]
    return pl.pallas_call(
        _matmul_add_kern,
        grid=(T // TB,),
        in_specs=[
            pl.BlockSpec((TB, D), lambda i: (i, 0)),
            pl.BlockSpec((D, N), lambda i: (0, 0)),
            pl.BlockSpec((TB, N), lambda i: (i, 0)),
        ],
        out_specs=pl.BlockSpec((TB, N), lambda i: (i, 0)),
        out_shape=jax.ShapeDtypeStruct((T, N), jnp.float32),
    )(a, w, r)


def _mlp_kern(x_ref, s_ref, wg_ref, wu_ref, wd_ref, o_ref):
    fb = pl.program_id(1)
    s = s_ref[:]
    half = TB // 2
    for h in range(2):
        rows = pl.ds(h * half, half)
        x = x_ref[rows, :]
        var = jnp.mean(jnp.square(x), axis=-1, keepdims=True)
        xn = (x * lax.rsqrt(var + EPS)) * s
        g = jnp.dot(xn, wg_ref[:], preferred_element_type=jnp.float32)
        u = jnp.dot(xn, wu_ref[:], preferred_element_type=jnp.float32)
        t = (g * jax.nn.sigmoid(g)) * u
        part = jnp.dot(t, wd_ref[:], preferred_element_type=jnp.float32)

        @pl.when(fb == 0)
        def _():
            o_ref[rows, :] = x + part

        @pl.when(fb != 0)
        def _():
            o_ref[rows, :] = o_ref[rows, :] + part


def _mlp_residual(x, s, wg, wu, wd):
    T = x.shape[0]
    return pl.pallas_call(
        _mlp_kern,
        grid=(T // TB, F // FB),
        in_specs=[
            pl.BlockSpec((TB, D), lambda i, j: (i, 0)),
            pl.BlockSpec((1, D), lambda i, j: (0, 0)),
            pl.BlockSpec((D, FB), lambda i, j: (0, j)),
            pl.BlockSpec((D, FB), lambda i, j: (0, j)),
            pl.BlockSpec((FB, D), lambda i, j: (j, 0)),
        ],
        out_specs=pl.BlockSpec((TB, D), lambda i, j: (i, 0)),
        out_shape=jax.ShapeDtypeStruct((T, D), jnp.float32),
    )(x, s, wg, wu, wd)


def _score_kern(proc_ref, s_ref, wg_ref, wu_ref, wd_ref, orig_ref,
                bce_ref, bcu_ref, g_ref, last_row, prev_s, pred_acc):
    i = pl.program_id(0)
    fb = pl.program_id(1)
    nfb = pl.num_programs(1)

    @pl.when((i == 0) & (fb == 0))
    def _():
        last_row[:] = jnp.zeros((1, D), jnp.float32)

    @pl.when(fb == 0)
    def _():
        proc = proc_ref[:]
        prev_s[:] = jnp.concatenate([last_row[:], proc[:TB - 1]], axis=0)
        last_row[:] = proc[TB - 1:TB]

    s = s_ref[:]
    half = TB // 2
    for h in range(2):
        rows = pl.ds(h * half, half)
        x = prev_s[rows, :]
        var = jnp.mean(jnp.square(x), axis=-1, keepdims=True)
        xn = (x * lax.rsqrt(var + EPS)) * s
        g = jnp.dot(xn, wg_ref[:], preferred_element_type=jnp.float32)
        u = jnp.dot(xn, wu_ref[:], preferred_element_type=jnp.float32)
        t = (g * jax.nn.sigmoid(g)) * u
        part = jnp.dot(t, wd_ref[:], preferred_element_type=jnp.float32)

        @pl.when(fb == 0)
        def _():
            pred_acc[rows, :] = part

        @pl.when(fb != 0)
        def _():
            pred_acc[rows, :] = pred_acc[rows, :] + part

    @pl.when(fb == nfb - 1)
    def _():
        pred = pred_acc[:]
        res = proc_ref[:] - orig_ref[:]
        d_st = jnp.sum(res * res, axis=-1, keepdims=True) * (1.0 / D)
        e = res - pred
        d_ch = jnp.sum(e * e, axis=-1, keepdims=True) * (1.0 / D)
        logit = bce_ref[0, 0] * d_st - bcu_ref[0, 0] * d_ch
        g_ref[:] = jax.nn.sigmoid(logit)


def _tn_scores(proc, s, wg, wu, wd, orig, bce, bcu):
    T = proc.shape[0]
    return pl.pallas_call(
        _score_kern,
        grid=(T // TB, F // FB),
        in_specs=[
            pl.BlockSpec((TB, D), lambda i, j: (i, 0)),
            pl.BlockSpec((1, D), lambda i, j: (0, 0)),
            pl.BlockSpec((D, FB), lambda i, j: (0, j)),
            pl.BlockSpec((D, FB), lambda i, j: (0, j)),
            pl.BlockSpec((FB, D), lambda i, j: (j, 0)),
            pl.BlockSpec((TB, D), lambda i, j: (i, 0)),
            pl.BlockSpec((1, 1), lambda i, j: (0, 0)),
            pl.BlockSpec((1, 1), lambda i, j: (0, 0)),
        ],
        out_specs=pl.BlockSpec((TB, 1), lambda i, j: (i, 0)),
        out_shape=jax.ShapeDtypeStruct((T, 1), jnp.float32),
        scratch_shapes=[
            pltpu.VMEM((1, D), jnp.float32),
            pltpu.VMEM((TB, D), jnp.float32),
            pltpu.VMEM((TB, D), jnp.float32),
        ],
    )(proc, s, wg, wu, wd, orig, bce, bcu)


_SC_LANES = 16


def _sc_gate_body(g_hbm, out_hbm, g_v, out_v, acc_ref, idx_ref, sem, *, T, k):
    L = _SC_LANES
    nslice = T // L
    info = plsc.get_sparse_core_info()
    nc = info.num_cores
    wid = lax.axis_index("s") * nc + lax.axis_index("c")
    nw = nc * info.num_subcores
    per_w = T // nw
    slices_per_w = per_w // L

    zero_v = jnp.zeros((L,), jnp.int32)
    one_v = jnp.ones((L,), jnp.int32)
    k_v = jnp.full((L,), k, jnp.int32)
    iota_v = jnp.arange(L, dtype=jnp.int32)
    lstep_v = jnp.full((L,), L, jnp.int32)

    pltpu.sync_copy(g_hbm, g_v)

    def lane_total_v(vec):
        tot = vec[0]
        for i in range(1, L):
            tot = tot + vec[i]
        return jnp.full((L,), tot, jnp.int32)

    def count_ge(mid_v):
        acc_ref[...] = zero_v

        def body(j, c):
            d = zero_v
            for u in range(4):
                b = lax.bitcast_convert_type(
                    g_v[pl.ds((j * 4 + u) * L, L)], jnp.int32)
                d = d + jnp.where(b >= mid_v, one_v, zero_v)
            acc_ref[...] = acc_ref[...] + d
            return c

        lax.fori_loop(0, nslice // 4, body, jnp.int32(0))
        return lane_total_v(acc_ref[...])

    lo_v = zero_v
    hi_v = jnp.full((L,), 0x3F800000, jnp.int32)
    for _ in range(31):
        mid_v = lo_v + ((hi_v - lo_v + one_v) >> one_v)
        ge_k = count_ge(mid_v) >= k_v
        lo_v = jnp.where(ge_k, mid_v, lo_v)
        hi_v = jnp.where(ge_k, hi_v, mid_v - one_v)
    thr_v = lo_v
    n_tie_sel_v = k_v - count_ge(thr_v + one_v)

    def count_tie_lt(m_v):
        acc_ref[...] = zero_v
        idx_ref[...] = iota_v

        def body(j, c):
            d = zero_v
            idx = idx_ref[...]
            for u in range(4):
                b = lax.bitcast_convert_type(
                    g_v[pl.ds((j * 4 + u) * L, L)], jnp.int32)
                m = (b == thr_v) & (idx < m_v)
                d = d + jnp.where(m, one_v, zero_v)
                idx = idx + lstep_v
            acc_ref[...] = acc_ref[...] + d
            idx_ref[...] = idx
            return c

        lax.fori_loop(0, nslice // 4, body, jnp.int32(0))
        return lane_total_v(acc_ref[...])

    tlo_v = zero_v
    thi_v = jnp.full((L,), T, jnp.int32)
    for _ in range(12):
        tmid_v = tlo_v + ((thi_v - tlo_v + one_v) >> one_v)
        ok = count_tie_lt(tmid_v) <= n_tie_sel_v
        tlo_v = jnp.where(ok, tmid_v, tlo_v)
        thi_v = jnp.where(ok, thi_v, tmid_v - one_v)
    m_v = tlo_v

    s0 = wid * slices_per_w
    for t in range(slices_per_w):
        vals = g_v[pl.ds((s0 + t) * L, L)]
        bits = lax.bitcast_convert_type(vals, jnp.int32)
        own_idx = jnp.full((L,), (s0 + t) * L, jnp.int32) + iota_v
        sel = (bits > thr_v) | ((bits == thr_v) & (own_idx < m_v))
        out_v[pl.ds(t * L, L)] = jnp.where(sel, vals,
                                           jnp.zeros((L,), jnp.float32))

    pltpu.sync_copy(out_v, out_hbm.at[pl.ds(wid * per_w, per_w)])


def _sc_gate(g, k):
    T = g.shape[0]
    mesh = plsc.VectorSubcoreMesh(core_axis_name="c", subcore_axis_name="s")
    fn = functools.partial(
        pl.kernel,
        mesh=mesh,
        out_type=jax.ShapeDtypeStruct((T,), jnp.float32),
        scratch_types=[
            pltpu.VMEM((T,), jnp.float32),
            pltpu.VMEM((T // 32,), jnp.float32),
            pltpu.VMEM((_SC_LANES,), jnp.int32),
            pltpu.VMEM((_SC_LANES,), jnp.int32),
            pltpu.SemaphoreType.DMA,
        ],
    )(functools.partial(_sc_gate_body, T=T, k=k))
    return fn(g)


def _combine_kern(gate_ref, orig_ref, proc_ref, o_ref):
    orig = orig_ref[:]
    o_ref[:] = orig + gate_ref[:] * (proc_ref[:] - orig)


def _combine(gate, orig, proc):
    T = orig.shape[0]
    return pl.pallas_call(
        _combine_kern,
        grid=(T // TB,),
        in_specs=[
            pl.BlockSpec((TB, 1), lambda i: (i, 0)),
            pl.BlockSpec((TB, D), lambda i: (i, 0)),
            pl.BlockSpec((TB, D), lambda i: (i, 0)),
        ],
        out_specs=pl.BlockSpec((TB, D), lambda i: (i, 0)),
        out_shape=jax.ShapeDtypeStruct((T, D), jnp.float32),
    )(gate, orig, proc)


def kernel(hidden_states, beta_ce, beta_cu, ln1, wq, wk, wv, wo, ln2,
           wg, wu, wd, tn_norm, tn_g, tn_u, tn_d):
    B, T, _ = hidden_states.shape
    x = hidden_states.reshape(T, D)

    q, k, v = _qkv(x, ln1.reshape(1, D), wq, wk, wv)
    o = _attention(q, k, v)
    h1 = _matmul_add(o, wo, x)

    proc = _mlp_residual(h1, ln2.reshape(1, D), wg, wu, wd)

    g_cont = _tn_scores(proc, tn_norm.reshape(1, D), tn_g, tn_u, tn_d,
                        x, beta_ce.reshape(1, 1), beta_cu.reshape(1, 1))

    kk = max(1, int(T * CAP))
    gate = _sc_gate(g_cont.reshape(T), kk)
    final = _combine(gate.reshape(T, 1), x, proc)
    return final.reshape(B, T, D)
